# baseline (device time: 110971 ns/iter reference)
import jax
import jax.numpy as jnp
from jax import lax
from jax.experimental import pallas as pl
from jax.experimental.pallas import tpu as pltpu

N_DEV = 4
SQ = 2048
SKV = 2048
D_MODEL = 1024
DH = 128
H_LOC = 8
BLK = 64
SCALE = 0.08838834764831843
QT = 512
N_HOPS = 2 * (N_DEV - 1)
HALF = SQ // 2
CH = HALF // N_DEV

_BF = jnp.bfloat16


def _rows(ring, c):
    return pl.ds(ring * HALF + c * CH, CH)


def _body(x_ref, wq_ref, k_hbm, v_hbm, wo_ref, out_ref,
          q_all, ctx_all, k_buf, v_buf, kv_sems,
          comm_rs, comm_ag, stage_rs, stage_ag,
          send_rs, recv_rs, send_ag, recv_ag):
    h = pl.program_id(0)
    d = lax.axis_index("i")

    def kv_copy(slot, head_idx):
        hd = d * H_LOC + head_idx
        ck = pltpu.make_async_copy(
            k_hbm.at[:, pl.ds(hd, 1), :], k_buf.at[slot], kv_sems.at[0, slot]
        )
        cv = pltpu.make_async_copy(
            v_hbm.at[:, pl.ds(hd, 1), :], v_buf.at[slot], kv_sems.at[1, slot]
        )
        return ck, cv

    @pl.when(h == 0)
    def _():
        ck, cv = kv_copy(0, 0)
        ck.start()
        cv.start()

    @pl.when(h < H_LOC - 1)
    def _():
        ck, cv = kv_copy((h + 1) % 2, h + 1)
        ck.start()
        cv.start()

    @pl.when(h == 0)
    def _():
        qf = jnp.dot(
            x_ref[...].astype(_BF), wq_ref[...],
            preferred_element_type=jnp.float32,
        )
        q_all[...] = (qf * SCALE).astype(_BF)

    slot = h % 2
    ck, cv = kv_copy(slot, h)
    ck.wait()
    cv.wait()
    k = k_buf[slot, :, 0, :].astype(_BF)
    v = v_buf[slot, :, 0, :].astype(_BF)

    q = q_all[:, pl.ds(h * DH, DH)]
    rb = lax.broadcasted_iota(jnp.int32, (QT, QT), 0) // BLK
    cb = lax.broadcasted_iota(jnp.int32, (QT, QT), 1) // BLK
    keep = cb <= rb
    for i in range(SQ // QT):
        r0 = i * QT
        kv = (i + 1) * QT
        qi = q[r0 : r0 + QT]
        s_diag = lax.dot_general(
            qi, k[r0:kv], (((1,), (1,)), ((), ())),
            preferred_element_type=jnp.float32,
        ).astype(_BF)
        w_diag = jnp.where(keep, jnp.exp(s_diag), jnp.array(0, _BF))
        if i == 0:
            l = jnp.sum(w_diag, axis=1, keepdims=True, dtype=jnp.float32)
            ctx = jnp.dot(
                w_diag, v[r0:kv], preferred_element_type=jnp.float32
            )
        else:
            s_pre = lax.dot_general(
                qi, k[:r0], (((1,), (1,)), ((), ())),
                preferred_element_type=jnp.float32,
            ).astype(_BF)
            w_pre = jnp.exp(s_pre)
            l = jnp.sum(
                w_pre, axis=1, keepdims=True, dtype=jnp.float32
            ) + jnp.sum(w_diag, axis=1, keepdims=True, dtype=jnp.float32)
            ctx = jnp.dot(
                w_pre, v[:r0], preferred_element_type=jnp.float32
            ) + jnp.dot(
                w_diag, v[r0:kv], preferred_element_type=jnp.float32
            )
        ctx = ctx / l
        ctx_all[r0 : r0 + QT, pl.ds(h * DH, DH)] = ctx.astype(_BF)

    @pl.when(h == H_LOC - 1)
    def _():
        wo = wo_ref[...]

        bar = pltpu.get_barrier_semaphore()
        for o in (1, 2, 3):
            pl.semaphore_signal(
                bar, inc=1,
                device_id=(jnp.mod(d + o, N_DEV),),
                device_id_type=pl.DeviceIdType.MESH,
            )
        pl.semaphore_wait(bar, 3)

        def proj(rows):
            return jnp.dot(
                ctx_all[rows, :], wo, preferred_element_type=jnp.float32
            )

        def rdma_rs(ring, o):
            return pltpu.make_async_remote_copy(
                src_ref=stage_rs.at[ring, o - 1],
                dst_ref=comm_rs.at[ring, 3 - o],
                send_sem=send_rs.at[ring, o - 1],
                recv_sem=recv_rs.at[ring, 3 - o],
                device_id=(jnp.mod(d + o, N_DEV),),
                device_id_type=pl.DeviceIdType.MESH,
            )

        def rdma_ag(ring, o):
            return pltpu.make_async_remote_copy(
                src_ref=stage_ag.at[ring],
                dst_ref=comm_ag.at[ring, 3 - o],
                send_sem=send_ag.at[ring, o - 1],
                recv_sem=recv_ag.at[ring, 3 - o],
                device_id=(jnp.mod(d + o, N_DEV),),
                device_id_type=pl.DeviceIdType.MESH,
            )

        for o in (1, 2, 3):
            t = jnp.mod(d + o, N_DEV)
            for ring in (0, 1):
                blk = proj(_rows(ring, t))
                out_ref[_rows(ring, t), :] = blk
                stage_rs[ring, o - 1] = blk.astype(_BF)
                rdma_rs(ring, o).start()
        for ring in (0, 1):
            out_ref[_rows(ring, d), :] = proj(_rows(ring, d))
        for o in (1, 2, 3):
            for ring in (0, 1):
                rdma_rs(ring, o).wait()
        for ring in (0, 1):
            out_ref[_rows(ring, d), :] += (
                comm_rs[ring, 0].astype(jnp.float32)
                + comm_rs[ring, 1].astype(jnp.float32)
                + comm_rs[ring, 2].astype(jnp.float32)
            )

        for ring in (0, 1):
            stage_ag[ring] = out_ref[_rows(ring, d), :].astype(_BF)
        for o in (1, 2, 3):
            for ring in (0, 1):
                rdma_ag(ring, o).start()
        for o in (1, 2, 3):
            for ring in (0, 1):
                rdma_ag(ring, o).wait()
        for o in (1, 2, 3):
            s = jnp.mod(d + o, N_DEV)
            for ring in (0, 1):
                out_ref[_rows(ring, s), :] = comm_ag[ring, o - 1].astype(
                    jnp.float32
                )


def kernel(x, Wq, K_ext, V_ext, Wo):
    x2 = x[0]
    y = pl.pallas_call(
        _body,
        grid=(H_LOC,),
        in_specs=[
            pl.BlockSpec((SQ, D_MODEL), lambda h: (0, 0)),
            pl.BlockSpec((D_MODEL, D_MODEL), lambda h: (0, 0)),
            pl.BlockSpec(memory_space=pl.ANY),
            pl.BlockSpec(memory_space=pl.ANY),
            pl.BlockSpec((D_MODEL, D_MODEL), lambda h: (0, 0)),
        ],
        out_specs=pl.BlockSpec((SQ, D_MODEL), lambda h: (0, 0)),
        out_shape=jax.ShapeDtypeStruct((SQ, D_MODEL), jnp.float32),
        scratch_shapes=[
            pltpu.VMEM((SQ, D_MODEL), _BF),
            pltpu.VMEM((SQ, D_MODEL), _BF),
            pltpu.VMEM((2, SKV, 1, DH), jnp.float32),
            pltpu.VMEM((2, SKV, 1, DH), jnp.float32),
            pltpu.SemaphoreType.DMA((2, 2)),
            pltpu.VMEM((2, 3, CH, D_MODEL), _BF),
            pltpu.VMEM((2, 3, CH, D_MODEL), _BF),
            pltpu.VMEM((2, 3, CH, D_MODEL), _BF),
            pltpu.VMEM((2, CH, D_MODEL), _BF),
            pltpu.SemaphoreType.DMA((2, 3)),
            pltpu.SemaphoreType.DMA((2, 3)),
            pltpu.SemaphoreType.DMA((2, 3)),
            pltpu.SemaphoreType.DMA((2, 3)),
        ],
        compiler_params=pltpu.CompilerParams(collective_id=0),
    )(x2, Wq.astype(_BF), K_ext[0], V_ext[0], Wo.astype(_BF))
    return y[None]


# device time: 107318 ns/iter; 1.0340x vs baseline; 1.0340x over previous
import jax
import jax.numpy as jnp
from jax import lax
from jax.experimental import pallas as pl
from jax.experimental.pallas import tpu as pltpu

N_DEV = 4
SQ = 2048
SKV = 2048
D_MODEL = 1024
DH = 128
H_LOC = 8
BLK = 64
SCALE = 0.08838834764831843
QT = 512
N_HOPS = 2 * (N_DEV - 1)
HALF = SQ // 2
CH = HALF // N_DEV

_BF = jnp.bfloat16


def _rows(ring, c):
    return pl.ds(ring * HALF + c * CH, CH)


def _body(x_ref, wq_ref, k_hbm, v_hbm, wo_ref, out_ref,
          q_all, ctx_all, k_buf, v_buf, kv_sems,
          comm_r, comm_l, stage_r, stage_l, send_sems, recv_sems):
    h = pl.program_id(0)
    d = lax.axis_index("i")

    def kv_copy(slot, head_idx):
        hd = d * H_LOC + head_idx
        ck = pltpu.make_async_copy(
            k_hbm.at[:, pl.ds(hd, 1), :], k_buf.at[slot], kv_sems.at[0, slot]
        )
        cv = pltpu.make_async_copy(
            v_hbm.at[:, pl.ds(hd, 1), :], v_buf.at[slot], kv_sems.at[1, slot]
        )
        return ck, cv

    @pl.when(h == 0)
    def _():
        ck, cv = kv_copy(0, 0)
        ck.start()
        cv.start()

    @pl.when(h < H_LOC - 1)
    def _():
        ck, cv = kv_copy((h + 1) % 2, h + 1)
        ck.start()
        cv.start()

    @pl.when(h == 0)
    def _():
        qf = jnp.dot(
            x_ref[...].astype(_BF), wq_ref[...],
            preferred_element_type=jnp.float32,
        )
        q_all[...] = (qf * SCALE).astype(_BF)

    slot = h % 2
    ck, cv = kv_copy(slot, h)
    ck.wait()
    cv.wait()
    k = k_buf[slot, :, 0, :].astype(_BF)
    v = v_buf[slot, :, 0, :].astype(_BF)

    q = q_all[:, pl.ds(h * DH, DH)]
    rb = lax.broadcasted_iota(jnp.int32, (QT, QT), 0) // BLK
    cb = lax.broadcasted_iota(jnp.int32, (QT, QT), 1) // BLK
    keep = cb <= rb
    for i in range(SQ // QT):
        r0 = i * QT
        kv = (i + 1) * QT
        qi = q[r0 : r0 + QT]
        s_diag = lax.dot_general(
            qi, k[r0:kv], (((1,), (1,)), ((), ())),
            preferred_element_type=jnp.float32,
        ).astype(_BF)
        w_diag = jnp.where(keep, jnp.exp(s_diag), jnp.array(0, _BF))
        if i == 0:
            l = jnp.sum(w_diag, axis=1, keepdims=True, dtype=jnp.float32)
            ctx = jnp.dot(
                w_diag, v[r0:kv], preferred_element_type=jnp.float32
            )
        else:
            s_pre = lax.dot_general(
                qi, k[:r0], (((1,), (1,)), ((), ())),
                preferred_element_type=jnp.float32,
            ).astype(_BF)
            w_pre = jnp.exp(s_pre)
            l = jnp.sum(
                w_pre, axis=1, keepdims=True, dtype=jnp.float32
            ) + jnp.sum(w_diag, axis=1, keepdims=True, dtype=jnp.float32)
            ctx = jnp.dot(
                w_pre, v[:r0], preferred_element_type=jnp.float32
            ) + jnp.dot(
                w_diag, v[r0:kv], preferred_element_type=jnp.float32
            )
        ctx = ctx / l
        ctx_all[r0 : r0 + QT, pl.ds(h * DH, DH)] = ctx.astype(_BF)

    @pl.when(h == H_LOC - 1)
    def _():
        wo = wo_ref[...]
        left = jnp.mod(d - 1, N_DEV)
        right = jnp.mod(d + 1, N_DEV)

        bar = pltpu.get_barrier_semaphore()
        for nbr in (left, right):
            pl.semaphore_signal(
                bar, inc=1,
                device_id=(nbr,), device_id_type=pl.DeviceIdType.MESH,
            )
        pl.semaphore_wait(bar, 2)

        def proj_store(ring, c):
            out_ref[_rows(ring, c), :] = jnp.dot(
                ctx_all[_rows(ring, c), :], wo,
                preferred_element_type=jnp.float32,
            )

        def rdma(src, t, ring):
            return pltpu.make_async_remote_copy(
                src_ref=src,
                dst_ref=(comm_r if ring == 0 else comm_l).at[t],
                send_sem=send_sems.at[ring, t],
                recv_sem=recv_sems.at[ring, t],
                device_id=(right if ring == 0 else left,),
                device_id_type=pl.DeviceIdType.MESH,
            )

        def hop(src_r, src_l, t):
            r = rdma(src_r, t, 0)
            l = rdma(src_l, t, 1)
            r.start()
            l.start()
            if t == 0:
                for o in (1, 2, 3):
                    proj_store(0, jnp.mod(d - o, N_DEV))
                    proj_store(1, jnp.mod(d + o, N_DEV))
            r.wait()
            l.wait()

        proj_store(0, d)
        proj_store(1, d)

        for t in range(N_DEV - 1):
            stage_r[...] = out_ref[_rows(0, jnp.mod(d - t, N_DEV)), :].astype(
                _BF
            )
            stage_l[...] = out_ref[_rows(1, jnp.mod(d + t, N_DEV)), :].astype(
                _BF
            )
            hop(stage_r, stage_l, t)
            out_ref[_rows(0, jnp.mod(d - 1 - t, N_DEV)), :] += comm_r[
                t
            ].astype(jnp.float32)
            out_ref[_rows(1, jnp.mod(d + 1 + t, N_DEV)), :] += comm_l[
                t
            ].astype(jnp.float32)

        for s in range(N_DEV - 1):
            t = (N_DEV - 1) + s
            if s == 0:
                stage_r[...] = out_ref[
                    _rows(0, jnp.mod(d + 1, N_DEV)), :
                ].astype(_BF)
                stage_l[...] = out_ref[
                    _rows(1, jnp.mod(d - 1, N_DEV)), :
                ].astype(_BF)
                src_r, src_l = stage_r, stage_l
            else:
                src_r, src_l = comm_r.at[t - 1], comm_l.at[t - 1]
            hop(src_r, src_l, t)
            out_ref[_rows(0, jnp.mod(d - s, N_DEV)), :] = comm_r[t].astype(
                jnp.float32
            )
            out_ref[_rows(1, jnp.mod(d + s, N_DEV)), :] = comm_l[t].astype(
                jnp.float32
            )


def kernel(x, Wq, K_ext, V_ext, Wo):
    x2 = x[0]
    y = pl.pallas_call(
        _body,
        grid=(H_LOC,),
        in_specs=[
            pl.BlockSpec((SQ, D_MODEL), lambda h: (0, 0)),
            pl.BlockSpec((D_MODEL, D_MODEL), lambda h: (0, 0)),
            pl.BlockSpec(memory_space=pl.ANY),
            pl.BlockSpec(memory_space=pl.ANY),
            pl.BlockSpec((D_MODEL, D_MODEL), lambda h: (0, 0)),
        ],
        out_specs=pl.BlockSpec((SQ, D_MODEL), lambda h: (0, 0)),
        out_shape=jax.ShapeDtypeStruct((SQ, D_MODEL), jnp.float32),
        scratch_shapes=[
            pltpu.VMEM((SQ, D_MODEL), _BF),
            pltpu.VMEM((SQ, D_MODEL), _BF),
            pltpu.VMEM((2, SKV, 1, DH), jnp.float32),
            pltpu.VMEM((2, SKV, 1, DH), jnp.float32),
            pltpu.SemaphoreType.DMA((2, 2)),
            pltpu.VMEM((N_HOPS, CH, D_MODEL), _BF),
            pltpu.VMEM((N_HOPS, CH, D_MODEL), _BF),
            pltpu.VMEM((CH, D_MODEL), _BF),
            pltpu.VMEM((CH, D_MODEL), _BF),
            pltpu.SemaphoreType.DMA((2, N_HOPS)),
            pltpu.SemaphoreType.DMA((2, N_HOPS)),
        ],
        compiler_params=pltpu.CompilerParams(collective_id=0),
    )(x2, Wq.astype(_BF), K_ext[0], V_ext[0], Wo.astype(_BF))
    return y[None]
